# Initial kernel scaffold; baseline (speedup 1.0000x reference)
import jax
import jax.numpy as jnp
from jax.experimental import pallas as pl


def kernel(x, task_ids, task_matrix, W_task, b_task, w_gate, W1, b1, W2, b2):
    raise NotImplementedError("write your pallas kernel here")


# speedup vs baseline: 1.0095x; 1.0095x over previous
import jax
import jax.numpy as jnp
from jax import lax
from jax.experimental import pallas as pl
from jax.experimental.pallas import tpu as pltpu

B = 2048
D = 768
H = 768
E = 8
NUM_TASKS = 10
T_DIM = 512
T_DIM2 = 64
LOSS_COEF = 1e-2
EPS = 2.220446049250313e-16

BLK = 256
NBB = B // BLK


def _gate_kernel(tid_ref, x_ref, tm_ref, wt_ref, bt_ref, wg_ref,
                 gates_ref, loss_ref):
    proj = jnp.dot(tm_ref[...], wt_ref[...],
                   preferred_element_type=jnp.float32) + bt_ref[...]
    tid = tid_ref[...]
    t_iota = lax.broadcasted_iota(jnp.int32, (B, NUM_TASKS), 1)
    oh_task = (tid == t_iota).astype(jnp.float32)
    temb = jnp.dot(oh_task, proj, preferred_element_type=jnp.float32)

    wg = wg_ref[...]
    logits = (jnp.dot(x_ref[...], wg[:D, :], preferred_element_type=jnp.float32)
              + jnp.dot(temb, wg[D:, :], preferred_element_type=jnp.float32))

    e_iota = lax.broadcasted_iota(jnp.int32, (B, E), 1)
    m0 = jnp.max(logits, axis=1, keepdims=True)
    i0 = jnp.min(jnp.where(logits == m0, e_iota, E), axis=1, keepdims=True)
    masked = jnp.where(e_iota == i0, -jnp.inf, logits)
    m1 = jnp.max(masked, axis=1, keepdims=True)
    i1 = jnp.min(jnp.where(masked == m1, e_iota, E), axis=1, keepdims=True)
    d = jnp.exp(m1 - m0)
    g0 = 1.0 / (1.0 + d)
    g1 = d / (1.0 + d)
    oh0 = (e_iota == i0).astype(jnp.float32)
    oh1 = (e_iota == i1).astype(jnp.float32)
    gates = oh0 * g0 + oh1 * g1
    gates_ref[...] = gates

    imp = jnp.sum(gates, axis=0, keepdims=True)
    load = jnp.sum(oh0 + oh1, axis=0, keepdims=True)
    mean_i = jnp.sum(imp, axis=1, keepdims=True) / E
    var_i = jnp.sum((imp - mean_i) ** 2, axis=1, keepdims=True) / (E - 1)
    cv_i = var_i / (mean_i * mean_i + 1e-10)
    mean_l = jnp.sum(load, axis=1, keepdims=True) / E
    var_l = jnp.sum((load - mean_l) ** 2, axis=1, keepdims=True) / (E - 1)
    cv_l = var_l / (mean_l * mean_l + 1e-10)
    loss_ref[...] = (cv_i + cv_l) * LOSS_COEF


def _ffn_kernel(gates_ref, x_ref, w1_ref, b1_ref, w2_ref, b2_ref,
                y_ref, acc_ref):
    j = pl.program_id(1)
    xb = x_ref[...]
    h = jnp.maximum(
        jnp.dot(xb, w1_ref[0], preferred_element_type=jnp.float32)
        + b1_ref[0], 0.0)
    out = (jnp.dot(h, w2_ref[0], preferred_element_type=jnp.float32)
           + b2_ref[0])
    onehot = (lax.broadcasted_iota(jnp.int32, (E, 1), 0) == j).astype(jnp.float32)
    g = jnp.dot(gates_ref[...], onehot, preferred_element_type=jnp.float32)
    contrib = g * jnp.exp(out)

    @pl.when(j == 0)
    def _():
        acc_ref[...] = contrib

    @pl.when(j > 0)
    def _():
        acc_ref[...] += contrib

    @pl.when(j == E - 1)
    def _():
        a = acc_ref[...]
        y_ref[...] = jnp.log(jnp.where(a == 0.0, EPS, a))


def kernel(x, task_ids, task_matrix, W_task, b_task, w_gate, W1, b1, W2, b2):
    tid = task_ids.reshape(B, 1).astype(jnp.int32)
    bt = b_task.reshape(1, T_DIM2)

    gates, loss = pl.pallas_call(
        _gate_kernel,
        out_shape=[
            jax.ShapeDtypeStruct((B, E), jnp.float32),
            jax.ShapeDtypeStruct((1, 1), jnp.float32),
        ],
    )(tid, x, task_matrix, W_task, bt, w_gate)

    b1r = b1.reshape(E, 1, H)
    b2r = b2.reshape(E, 1, D)
    y = pl.pallas_call(
        _ffn_kernel,
        grid=(NBB, E),
        in_specs=[
            pl.BlockSpec((BLK, E), lambda i, j: (i, 0)),
            pl.BlockSpec((BLK, D), lambda i, j: (i, 0)),
            pl.BlockSpec((1, D, H), lambda i, j: (j, 0, 0)),
            pl.BlockSpec((1, 1, H), lambda i, j: (j, 0, 0)),
            pl.BlockSpec((1, H, D), lambda i, j: (j, 0, 0)),
            pl.BlockSpec((1, 1, D), lambda i, j: (j, 0, 0)),
        ],
        out_specs=pl.BlockSpec((BLK, D), lambda i, j: (i, 0)),
        out_shape=jax.ShapeDtypeStruct((B, D), jnp.float32),
        scratch_shapes=[pltpu.VMEM((BLK, D), jnp.float32)],
    )(gates, x, W1, b1r, W2, b2r)

    return y, loss.reshape(())


# speedup vs baseline: 1.2568x; 1.2449x over previous
import functools

import jax
import jax.numpy as jnp
from jax import lax
from jax.experimental import pallas as pl
from jax.experimental.pallas import tpu as pltpu
from jax.experimental.pallas import tpu_sc as plsc

B = 2048
D = 768
H = 768
E = 8
NUM_TASKS = 10
T_DIM = 512
T_DIM2 = 64
LOSS_COEF = 1e-2
EPS = 2.220446049250313e-16

BLK = 256
NB_MAX = (2 * B) // BLK + E
PAD = NB_MAX * BLK

NC = 2
NS = 16
NW = NC * NS
CH = B // NW


def _cumsum_rows(a):
    n = a.shape[0]
    s = 1
    while s < n:
        z = jnp.zeros((s, a.shape[1]), a.dtype)
        a = a + jnp.concatenate([z, a[: n - s, :]], axis=0)
        s *= 2
    return a


def _gate_kernel(tid_ref, x_ref, tm_ref, wt_ref, bt_ref, wg_ref,
                 pos0_ref, pos1_ref, g0_ref, g1_ref, blk_e_ref, loss_ref):
    proj = jnp.dot(tm_ref[...], wt_ref[...],
                   preferred_element_type=jnp.float32) + bt_ref[...]
    tid = tid_ref[...]
    t_iota = lax.broadcasted_iota(jnp.int32, (B, NUM_TASKS), 1)
    oh_task = (tid == t_iota).astype(jnp.float32)
    temb = jnp.dot(oh_task, proj, preferred_element_type=jnp.float32)

    wg = wg_ref[...]
    logits = (jnp.dot(x_ref[...], wg[:D, :], preferred_element_type=jnp.float32)
              + jnp.dot(temb, wg[D:, :], preferred_element_type=jnp.float32))

    e_iota = lax.broadcasted_iota(jnp.int32, (B, E), 1)
    m0 = jnp.max(logits, axis=1, keepdims=True)
    i0 = jnp.min(jnp.where(logits == m0, e_iota, E), axis=1, keepdims=True)
    masked = jnp.where(e_iota == i0, -jnp.inf, logits)
    m1 = jnp.max(masked, axis=1, keepdims=True)
    i1 = jnp.min(jnp.where(masked == m1, e_iota, E), axis=1, keepdims=True)
    d = jnp.exp(m1 - m0)
    g0 = 1.0 / (1.0 + d)
    g1 = d / (1.0 + d)
    g0_ref[...] = g0
    g1_ref[...] = g1
    oh0 = (e_iota == i0).astype(jnp.float32)
    oh1 = (e_iota == i1).astype(jnp.float32)

    gates = oh0 * g0 + oh1 * g1
    imp = jnp.sum(gates, axis=0, keepdims=True)
    cnt0 = jnp.sum(oh0, axis=0, keepdims=True)
    cnt1 = jnp.sum(oh1, axis=0, keepdims=True)
    load = cnt0 + cnt1
    mean_i = jnp.sum(imp, axis=1, keepdims=True) / E
    var_i = jnp.sum((imp - mean_i) ** 2, axis=1, keepdims=True) / (E - 1)
    cv_i = var_i / (mean_i * mean_i + 1e-10)
    mean_l = jnp.sum(load, axis=1, keepdims=True) / E
    var_l = jnp.sum((load - mean_l) ** 2, axis=1, keepdims=True) / (E - 1)
    cv_l = var_l / (mean_l * mean_l + 1e-10)
    loss_ref[...] = (cv_i + cv_l) * LOSS_COEF

    cum0 = _cumsum_rows(oh0)
    cum1 = _cumsum_rows(oh1)
    rank0 = cum0 - oh0
    rank1 = cum1 - oh1

    cnt_i = (cnt0 + cnt1).astype(jnp.int32)
    padded = ((cnt_i + (BLK - 1)) // BLK) * BLK
    inc = padded
    s = 1
    while s < E:
        z = jnp.zeros((1, s), jnp.int32)
        inc = inc + jnp.concatenate([z, inc[:, : E - s]], axis=1)
        s *= 2
    offs = (inc - padded).astype(jnp.float32)

    pos0 = jnp.sum(oh0 * (offs + rank0), axis=1, keepdims=True)
    pos1 = jnp.sum(oh1 * (offs + cnt0 + rank1), axis=1, keepdims=True)
    pos0_ref[...] = pos0.astype(jnp.int32)
    pos1_ref[...] = pos1.astype(jnp.int32)

    offs_blk = (inc - padded) // BLK
    jb = lax.broadcasted_iota(jnp.int32, (NB_MAX, E), 0)
    blk_e_ref[...] = jnp.sum(
        (jb >= offs_blk).astype(jnp.int32), axis=1, keepdims=True) - 1


def _gating(x, task_ids, task_matrix, W_task, b_task, w_gate):
    tid = task_ids.reshape(B, 1).astype(jnp.int32)
    bt = b_task.reshape(1, T_DIM2)
    return pl.pallas_call(
        _gate_kernel,
        out_shape=[
            jax.ShapeDtypeStruct((B, 1), jnp.int32),
            jax.ShapeDtypeStruct((B, 1), jnp.int32),
            jax.ShapeDtypeStruct((B, 1), jnp.float32),
            jax.ShapeDtypeStruct((B, 1), jnp.float32),
            jax.ShapeDtypeStruct((NB_MAX, 1), jnp.int32),
            jax.ShapeDtypeStruct((1, 1), jnp.float32),
        ],
    )(tid, x, task_matrix, W_task, bt, w_gate)


@functools.cache
def _sc_kernels():
    mesh = plsc.VectorSubcoreMesh(
        core_axis_name="c", subcore_axis_name="s",
        num_cores=NC, num_subcores=NS)
    sc_scratch = [
        pltpu.VMEM((CH,), jnp.int32),
        pltpu.VMEM((CH,), jnp.int32),
        pltpu.VMEM((CH, D), jnp.float32),
        pltpu.SemaphoreType.DMA,
    ]

    @functools.partial(
        pl.kernel,
        out_type=jax.ShapeDtypeStruct((PAD, D), jnp.float32),
        mesh=mesh,
        scratch_types=sc_scratch,
    )
    def _dispatch(x_hbm, pos0_hbm, pos1_hbm, xd_hbm,
                  idx0_v, idx1_v, rows_v, sem):
        wid = lax.axis_index("s") * NC + lax.axis_index("c")
        base = wid * CH
        pltpu.sync_copy(pos0_hbm.at[pl.ds(base, CH)], idx0_v)
        pltpu.sync_copy(pos1_hbm.at[pl.ds(base, CH)], idx1_v)
        pltpu.sync_copy(x_hbm.at[pl.ds(base, CH)], rows_v)
        pltpu.async_copy(rows_v, xd_hbm.at[idx0_v], sem).wait()
        pltpu.async_copy(rows_v, xd_hbm.at[idx1_v], sem).wait()

    @functools.partial(
        pl.kernel,
        out_type=[
            jax.ShapeDtypeStruct((B, D), jnp.float32),
            jax.ShapeDtypeStruct((B, D), jnp.float32),
        ],
        mesh=mesh,
        scratch_types=sc_scratch,
    )
    def _combine(c_hbm, pos0_hbm, pos1_hbm, c0_hbm, c1_hbm,
                 idx0_v, idx1_v, rows_v, sem):
        wid = lax.axis_index("s") * NC + lax.axis_index("c")
        base = wid * CH
        pltpu.sync_copy(pos0_hbm.at[pl.ds(base, CH)], idx0_v)
        pltpu.sync_copy(pos1_hbm.at[pl.ds(base, CH)], idx1_v)
        pltpu.async_copy(c_hbm.at[idx0_v], rows_v, sem).wait()
        pltpu.sync_copy(rows_v, c0_hbm.at[pl.ds(base, CH)])
        pltpu.async_copy(c_hbm.at[idx1_v], rows_v, sem).wait()
        pltpu.sync_copy(rows_v, c1_hbm.at[pl.ds(base, CH)])

    return _dispatch, _combine


def _ffn_kernel(be_ref, xd_ref, w1_ref, b1_ref, w2_ref, b2_ref, c_ref):
    del be_ref
    h = jnp.maximum(
        jnp.dot(xd_ref[...], w1_ref[0], preferred_element_type=jnp.float32)
        + b1_ref[0], 0.0)
    out = (jnp.dot(h, w2_ref[0], preferred_element_type=jnp.float32)
           + b2_ref[0])
    c_ref[...] = jnp.exp(out)


def _ffn(blk_expert, xd, W1, b1, W2, b2):
    b1r = b1.reshape(E, 1, H)
    b2r = b2.reshape(E, 1, D)
    grid_spec = pltpu.PrefetchScalarGridSpec(
        num_scalar_prefetch=1,
        grid=(NB_MAX,),
        in_specs=[
            pl.BlockSpec((BLK, D), lambda i, be: (i, 0)),
            pl.BlockSpec((1, D, H), lambda i, be: (be[i], 0, 0)),
            pl.BlockSpec((1, 1, H), lambda i, be: (be[i], 0, 0)),
            pl.BlockSpec((1, H, D), lambda i, be: (be[i], 0, 0)),
            pl.BlockSpec((1, 1, D), lambda i, be: (be[i], 0, 0)),
        ],
        out_specs=pl.BlockSpec((BLK, D), lambda i, be: (i, 0)),
    )
    return pl.pallas_call(
        _ffn_kernel,
        grid_spec=grid_spec,
        out_shape=jax.ShapeDtypeStruct((PAD, D), jnp.float32),
    )(blk_expert, xd, W1, b1r, W2, b2r)


def _final_kernel(g0_ref, g1_ref, c0_ref, c1_ref, y_ref):
    def _r(v):
        return v.astype(jnp.bfloat16).astype(jnp.float32)
    comb = _r(g0_ref[...]) * _r(c0_ref[...]) + _r(g1_ref[...]) * _r(c1_ref[...])
    y_ref[...] = jnp.log(jnp.where(comb == 0.0, EPS, comb))


def kernel(x, task_ids, task_matrix, W_task, b_task, w_gate, W1, b1, W2, b2):
    pos0, pos1, g0, g1, blk_expert, loss = _gating(
        x, task_ids, task_matrix, W_task, b_task, w_gate)
    dispatch, combine = _sc_kernels()
    p0 = pos0.reshape(B)
    p1 = pos1.reshape(B)
    xd = dispatch(x, p0, p1)
    c = _ffn(blk_expert.reshape(NB_MAX), xd, W1, b1, W2, b2)
    c0, c1 = combine(c, p0, p1)
    y = pl.pallas_call(
        _final_kernel,
        out_shape=jax.ShapeDtypeStruct((B, D), jnp.float32),
    )(g0, g1, c0, c1)
    return y, loss.reshape(())


# speedup vs baseline: 1.2630x; 1.0050x over previous
import functools

import jax
import jax.numpy as jnp
from jax import lax
from jax.experimental import pallas as pl
from jax.experimental.pallas import tpu as pltpu
from jax.experimental.pallas import tpu_sc as plsc

B = 2048
D = 768
H = 768
E = 8
NUM_TASKS = 10
T_DIM = 512
T_DIM2 = 64
LOSS_COEF = 1e-2
EPS = 2.220446049250313e-16

BLK = 256
NB_MAX = (2 * B) // BLK + E
PAD = NB_MAX * BLK

NC = 2
NS = 16
NW = NC * NS
CH = B // NW


def _cumsum_rows(a):
    n = a.shape[0]
    s = 1
    while s < n:
        z = jnp.zeros((s, a.shape[1]), a.dtype)
        a = a + jnp.concatenate([z, a[: n - s, :]], axis=0)
        s *= 2
    return a


def _gate_kernel(tid_ref, x_ref, tm_ref, wt_ref, bt_ref, wg_ref,
                 pos0_ref, pos1_ref, g0_ref, g1_ref, blk_e_ref, loss_ref):
    proj = jnp.dot(tm_ref[...], wt_ref[...],
                   preferred_element_type=jnp.float32) + bt_ref[...]
    tid = tid_ref[...]
    t_iota = lax.broadcasted_iota(jnp.int32, (B, NUM_TASKS), 1)
    oh_task = (tid == t_iota).astype(jnp.float32)
    temb = jnp.dot(oh_task, proj, preferred_element_type=jnp.float32)

    wg = wg_ref[...]
    logits = (jnp.dot(x_ref[...], wg[:D, :], preferred_element_type=jnp.float32)
              + jnp.dot(temb, wg[D:, :], preferred_element_type=jnp.float32))

    e_iota = lax.broadcasted_iota(jnp.int32, (B, E), 1)
    m0 = jnp.max(logits, axis=1, keepdims=True)
    i0 = jnp.min(jnp.where(logits == m0, e_iota, E), axis=1, keepdims=True)
    masked = jnp.where(e_iota == i0, -jnp.inf, logits)
    m1 = jnp.max(masked, axis=1, keepdims=True)
    i1 = jnp.min(jnp.where(masked == m1, e_iota, E), axis=1, keepdims=True)
    d = jnp.exp(m1 - m0)
    g0 = 1.0 / (1.0 + d)
    g1 = d / (1.0 + d)
    g0_ref[...] = g0
    g1_ref[...] = g1
    oh0 = (e_iota == i0).astype(jnp.float32)
    oh1 = (e_iota == i1).astype(jnp.float32)

    gates = oh0 * g0 + oh1 * g1
    imp = jnp.sum(gates, axis=0, keepdims=True)
    cnt0 = jnp.sum(oh0, axis=0, keepdims=True)
    cnt1 = jnp.sum(oh1, axis=0, keepdims=True)
    load = cnt0 + cnt1
    mean_i = jnp.sum(imp, axis=1, keepdims=True) / E
    var_i = jnp.sum((imp - mean_i) ** 2, axis=1, keepdims=True) / (E - 1)
    cv_i = var_i / (mean_i * mean_i + 1e-10)
    mean_l = jnp.sum(load, axis=1, keepdims=True) / E
    var_l = jnp.sum((load - mean_l) ** 2, axis=1, keepdims=True) / (E - 1)
    cv_l = var_l / (mean_l * mean_l + 1e-10)
    loss_ref[...] = (cv_i + cv_l) * LOSS_COEF

    cum0 = _cumsum_rows(oh0)
    cum1 = _cumsum_rows(oh1)
    rank0 = cum0 - oh0
    rank1 = cum1 - oh1

    cnt_i = (cnt0 + cnt1).astype(jnp.int32)
    padded = ((cnt_i + (BLK - 1)) // BLK) * BLK
    inc = padded
    s = 1
    while s < E:
        z = jnp.zeros((1, s), jnp.int32)
        inc = inc + jnp.concatenate([z, inc[:, : E - s]], axis=1)
        s *= 2
    offs = (inc - padded).astype(jnp.float32)

    pos0 = jnp.sum(oh0 * (offs + rank0), axis=1, keepdims=True)
    pos1 = jnp.sum(oh1 * (offs + cnt0 + rank1), axis=1, keepdims=True)
    pos0_ref[...] = pos0.astype(jnp.int32)
    pos1_ref[...] = pos1.astype(jnp.int32)

    offs_blk = (inc - padded) // BLK
    jb = lax.broadcasted_iota(jnp.int32, (NB_MAX, E), 0)
    blk_e_ref[...] = jnp.sum(
        (jb >= offs_blk).astype(jnp.int32), axis=1, keepdims=True) - 1


def _gating(x, task_ids, task_matrix, W_task, b_task, w_gate):
    tid = task_ids.reshape(B, 1).astype(jnp.int32)
    bt = b_task.reshape(1, T_DIM2)
    return pl.pallas_call(
        _gate_kernel,
        out_shape=[
            jax.ShapeDtypeStruct((B, 1), jnp.int32),
            jax.ShapeDtypeStruct((B, 1), jnp.int32),
            jax.ShapeDtypeStruct((B, 1), jnp.float32),
            jax.ShapeDtypeStruct((B, 1), jnp.float32),
            jax.ShapeDtypeStruct((NB_MAX, 1), jnp.int32),
            jax.ShapeDtypeStruct((1, 1), jnp.float32),
        ],
    )(tid, x, task_matrix, W_task, bt, w_gate)


@functools.cache
def _sc_kernels():
    mesh = plsc.VectorSubcoreMesh(
        core_axis_name="c", subcore_axis_name="s",
        num_cores=NC, num_subcores=NS)
    sc_scratch = [
        pltpu.VMEM((CH,), jnp.int32),
        pltpu.VMEM((CH,), jnp.int32),
        pltpu.VMEM((CH, D), jnp.float32),
        pltpu.SemaphoreType.DMA,
    ]

    @functools.partial(
        pl.kernel,
        out_type=jax.ShapeDtypeStruct((PAD, D), jnp.float32),
        mesh=mesh,
        scratch_types=sc_scratch,
    )
    def _dispatch(x_hbm, pos0_hbm, pos1_hbm, xd_hbm,
                  idx0_v, idx1_v, rows_v, sem):
        wid = lax.axis_index("s") * NC + lax.axis_index("c")
        base = wid * CH
        pltpu.sync_copy(pos0_hbm.at[pl.ds(base, CH)], idx0_v)
        pltpu.sync_copy(pos1_hbm.at[pl.ds(base, CH)], idx1_v)
        pltpu.sync_copy(x_hbm.at[pl.ds(base, CH)], rows_v)
        pltpu.async_copy(rows_v, xd_hbm.at[idx0_v], sem).wait()
        pltpu.async_copy(rows_v, xd_hbm.at[idx1_v], sem).wait()

    @functools.partial(
        pl.kernel,
        out_type=[
            jax.ShapeDtypeStruct((B, D), jnp.float32),
            jax.ShapeDtypeStruct((B, D), jnp.float32),
        ],
        mesh=mesh,
        scratch_types=sc_scratch,
    )
    def _combine(c_hbm, pos0_hbm, pos1_hbm, c0_hbm, c1_hbm,
                 idx0_v, idx1_v, rows_v, sem):
        wid = lax.axis_index("s") * NC + lax.axis_index("c")
        base = wid * CH
        pltpu.sync_copy(pos0_hbm.at[pl.ds(base, CH)], idx0_v)
        pltpu.sync_copy(pos1_hbm.at[pl.ds(base, CH)], idx1_v)
        pltpu.async_copy(c_hbm.at[idx0_v], rows_v, sem).wait()
        pltpu.sync_copy(rows_v, c0_hbm.at[pl.ds(base, CH)])
        pltpu.async_copy(c_hbm.at[idx1_v], rows_v, sem).wait()
        pltpu.sync_copy(rows_v, c1_hbm.at[pl.ds(base, CH)])

    return _dispatch, _combine


def _ffn_kernel(be_ref, xd_ref, w1_ref, b1_ref, w2_ref, b2_ref, c_ref):
    e = be_ref[pl.program_id(0)]
    h = jnp.maximum(
        jnp.dot(xd_ref[...], w1_ref[e], preferred_element_type=jnp.float32)
        + b1_ref[e], 0.0)
    out = (jnp.dot(h, w2_ref[e], preferred_element_type=jnp.float32)
           + b2_ref[e])
    c_ref[...] = jnp.exp(out)


def _ffn(blk_expert, xd, W1, b1, W2, b2):
    b1r = b1.reshape(E, 1, H)
    b2r = b2.reshape(E, 1, D)
    grid_spec = pltpu.PrefetchScalarGridSpec(
        num_scalar_prefetch=1,
        grid=(NB_MAX,),
        in_specs=[
            pl.BlockSpec((BLK, D), lambda i, be: (i, 0)),
            pl.BlockSpec((E, D, H), lambda i, be: (0, 0, 0)),
            pl.BlockSpec((E, 1, H), lambda i, be: (0, 0, 0)),
            pl.BlockSpec((E, H, D), lambda i, be: (0, 0, 0)),
            pl.BlockSpec((E, 1, D), lambda i, be: (0, 0, 0)),
        ],
        out_specs=pl.BlockSpec((BLK, D), lambda i, be: (i, 0)),
    )
    return pl.pallas_call(
        _ffn_kernel,
        grid_spec=grid_spec,
        out_shape=jax.ShapeDtypeStruct((PAD, D), jnp.float32),
    )(blk_expert, xd, W1, b1r, W2, b2r)


def _final_kernel(g0_ref, g1_ref, c0_ref, c1_ref, y_ref):
    def _r(v):
        return v.astype(jnp.bfloat16).astype(jnp.float32)
    comb = _r(g0_ref[...]) * _r(c0_ref[...]) + _r(g1_ref[...]) * _r(c1_ref[...])
    y_ref[...] = jnp.log(jnp.where(comb == 0.0, EPS, comb))


def kernel(x, task_ids, task_matrix, W_task, b_task, w_gate, W1, b1, W2, b2):
    pos0, pos1, g0, g1, blk_expert, loss = _gating(
        x, task_ids, task_matrix, W_task, b_task, w_gate)
    dispatch, combine = _sc_kernels()
    p0 = pos0.reshape(B)
    p1 = pos1.reshape(B)
    xd = dispatch(x, p0, p1)
    c = _ffn(blk_expert.reshape(NB_MAX), xd, W1, b1, W2, b2)
    c0, c1 = combine(c, p0, p1)
    y = pl.pallas_call(
        _final_kernel,
        grid=(B // BLK,),
        in_specs=[
            pl.BlockSpec((BLK, 1), lambda i: (i, 0)),
            pl.BlockSpec((BLK, 1), lambda i: (i, 0)),
            pl.BlockSpec((BLK, D), lambda i: (i, 0)),
            pl.BlockSpec((BLK, D), lambda i: (i, 0)),
        ],
        out_specs=pl.BlockSpec((BLK, D), lambda i: (i, 0)),
        out_shape=jax.ShapeDtypeStruct((B, D), jnp.float32),
    )(g0, g1, c0, c1)
    return y, loss.reshape(())
